# baseline (speedup 1.0000x reference)
import jax
import jax.numpy as jnp
import numpy as np
from jax import lax
from jax.experimental import pallas as pl
from jax.experimental.pallas import tpu as pltpu, tpu_sc as plsc

_LEVEL_HW = [(16, 16), (32, 32), (64, 64), (128, 128)]
_LEVEL_FILTER = [0.25, 0.5, 1.0, 1.0]
_LAYER_FILTER = [1.0, 0.8, 0.6, 0.6, 0.4, 0.2]

_N = 21760
_CH = _N // 16
_K_OUT = 21056
_KCH = _K_OUT // 16
_I32MIN = jnp.int32(-2147483648)
_NBINS = 2048
_U = 4


def _score_body(f_ref, u_ref, w_ref, o_ref):
    fm = f_ref[0]
    mod = fm + fm * u_ref[0].reshape(-1, 1)
    o_ref[...] = jnp.dot(mod, w_ref[...],
                         preferred_element_type=jnp.float32)[:, 0][None, None]


def _level_score(features, start, n, upa, W_cls, blk=2048):
    B, N, D = features.shape
    sblk = start // blk
    nblk = -(-(start + n - sblk * blk) // blk)
    cover = nblk * blk
    up3 = upa.reshape(B * nblk, 1, blk)
    out = pl.pallas_call(
        _score_body,
        grid=(B, nblk),
        in_specs=[pl.BlockSpec((1, blk, D), lambda b, i: (b, sblk + i, 0)),
                  pl.BlockSpec((1, 1, blk), lambda b, i: (b * nblk + i, 0, 0)),
                  pl.BlockSpec((D, 1), lambda b, i: (0, 0))],
        out_specs=pl.BlockSpec((1, 1, blk), lambda b, i: (b * nblk + i, 0, 0)),
        out_shape=jax.ShapeDtypeStruct((B * nblk, 1, blk), jnp.float32),
    )(features, up3, W_cls)
    off = start - sblk * blk
    return out.reshape(B, cover)[:, off:off + n]


def _digit(vals_f32, shift):
    u = plsc.bitcast(vals_f32, jnp.int32)
    m = lax.shift_right_arithmetic(u, 31)
    key = ~(u ^ (m | _I32MIN))
    return lax.shift_right_logical(key, shift) & (_NBINS - 1)


def _sc_body(sc_hbm, ss_hbm, ord_hbm, xx_hbm, yy_hbm, Ak, Ap, Bk, Bp, hist):
    wid = lax.axis_index("s") * 2 + lax.axis_index("c")

    @pl.when(wid < 4)
    def _():
        b = wid
        lane = lax.iota(jnp.int32, 16)
        lane_str = lane * _CH
        lane_h = lane * _NBINS

        pltpu.sync_copy(sc_hbm.at[pl.ds(b * _N, _N)], Ak)

        def radix_pass(shift, Ki, Pi, Ko, Po):
            zeros16 = jnp.zeros((16,), jnp.int32)

            @plsc.parallel_loop(0, 16 * _NBINS, step=16, unroll=8)
            def _z(o):
                hist[pl.ds(o, 16)] = zeros16

            ones = jnp.ones((16,), jnp.int32)

            def pa(c, _):
                for u in range(_U):
                    k = plsc.load_gather(Ki, [lane_str + (c * _U + u)])
                    d = _digit(k, shift)
                    plsc.addupdate_scatter(hist, [lane_h + d], ones)
                return 0
            lax.fori_loop(0, _CH // _U, pa, 0)

            def sc16(dc, carry):
                vs = [hist[pl.ds(l * _NBINS + dc * 16, 16)] for l in range(16)]
                a = jnp.zeros((16,), jnp.int32)
                accs = []
                for l in range(16):
                    accs.append(a)
                    a = a + vs[l]
                total = a
                g = carry + plsc.cumsum(total) - total
                for l in range(16):
                    hist[pl.ds(l * _NBINS + dc * 16, 16)] = accs[l] + g
                return carry + jnp.sum(total, axis=0)
            plsc.parallel_loop(0, _NBINS // 16, unroll=2,
                               carry=jnp.int32(0))(sc16)

            def pb(c, _):
                for u in range(_U):
                    idx = lane_str + (c * _U + u)
                    k = plsc.load_gather(Ki, [idx])
                    p = idx if Pi is None else plsc.load_gather(Pi, [idx])
                    d = _digit(k, shift)
                    h = lane_h + d
                    off = plsc.load_gather(hist, [h])
                    plsc.store_scatter(Ko, [off], k)
                    plsc.store_scatter(Po, [off], p)
                    plsc.store_scatter(hist, [h], off + 1)
                return 0
            lax.fori_loop(0, _CH // _U, pb, 0)

        radix_pass(0, Ak, None, Bk, Bp)
        radix_pass(11, Bk, Bp, Ak, Ap)
        radix_pass(22, Ak, Ap, Bk, Bp)

        zero = jnp.int32(0)

        def post(c, carry):
            for u in range(2):
                gsel, l0, l1, l2, l3 = carry
                cc = c * 2 + u
                s = Bk[pl.ds(cc * 16, 16)]
                t = Bp[pl.ds(cc * 16, 16)]
                ge1 = t >= 256
                ge2 = t >= 1280
                ge3 = t >= 5376
                sh = (ge1.astype(jnp.int32) + ge2.astype(jnp.int32)
                      + ge3.astype(jnp.int32)) << 3
                enc = lax.shift_left(jnp.ones((16,), jnp.int32), sh)
                scs = plsc.cumsum(enc)
                cnt = lax.shift_right_logical(scs - enc, sh) & 255
                lb = jnp.where(ge2, jnp.where(ge3, l3, l2),
                               jnp.where(ge1, l1, l0))
                rank = lb + cnt
                kv = jnp.where(ge2, jnp.where(ge3, 16384, 4096),
                               jnp.where(ge1, 512, 64))
                offv = jnp.where(ge2, jnp.where(ge3, 4672, 576),
                                 jnp.where(ge1, 64, 0))
                sel = rank < kv
                seli = sel.astype(jnp.int32)
                sx = plsc.cumsum(seli)
                gr = gsel + sx - seli
                gidx = jnp.where(sel, gr, zero)
                concat = offv + rank
                cidx = jnp.where(sel, concat, zero)
                plsc.store_scatter(Bk, [gidx], s, mask=sel)
                plsc.store_scatter(Ap, [gidx], concat, mask=sel)
                plsc.store_scatter(Ak, [cidx], plsc.bitcast(t, jnp.float32),
                                   mask=sel)
                tot = jnp.sum(enc, axis=0)
                nsel = jnp.sum(seli, axis=0)
                carry = (gsel + nsel,
                         l0 + (tot & 255),
                         l1 + (lax.shift_right_logical(tot, 8) & 255),
                         l2 + (lax.shift_right_logical(tot, 16) & 255),
                         l3 + (lax.shift_right_logical(tot, 24) & 255))
            return carry
        lax.fori_loop(0, _CH // 2, post, (zero, zero, zero, zero, zero))

        pltpu.sync_copy(Bk.at[pl.ds(0, _K_OUT)],
                        ss_hbm.at[pl.ds(b * _K_OUT, _K_OUT)])
        pltpu.sync_copy(Ap.at[pl.ds(0, _K_OUT)],
                        ord_hbm.at[pl.ds(b * _K_OUT, _K_OUT)])

        half = jnp.float32(0.5)
        onesv = jnp.ones((16,), jnp.int32)

        @plsc.parallel_loop(0, _KCH, unroll=4)
        def _xy(cc):
                tb = plsc.bitcast(Ak[pl.ds(cc * 16, 16)], jnp.int32)
                q = cc * 16 + lane
                qge1 = q >= 64
                qge2 = q >= 576
                qge3 = q >= 4672
                lvlq = (qge1.astype(jnp.int32) + qge2.astype(jnp.int32)
                        + qge3.astype(jnp.int32))
                startv = jnp.where(qge2, jnp.where(qge3, 5376, 1280),
                                   jnp.where(qge1, 256, 0))
                logw = 4 + lvlq
                uu = tb - startv
                jx = uu & (lax.shift_left(onesv, logw) - 1)
                iy = lax.shift_right_logical(uu, logw)
                invw = plsc.bitcast(lax.shift_left(127 - logw, 23),
                                    jnp.float32)
                xv = (jx.astype(jnp.float32) + half) * invw
                yv = (iy.astype(jnp.float32) + half) * invw
                Ap[pl.ds(cc * 16, 16)] = plsc.bitcast(xv, jnp.int32)
                Bp[pl.ds(cc * 16, 16)] = plsc.bitcast(yv, jnp.int32)

        pltpu.sync_copy(Ap.at[pl.ds(0, _K_OUT)],
                        xx_hbm.at[pl.ds(b * _K_OUT, _K_OUT)])
        pltpu.sync_copy(Bp.at[pl.ds(0, _K_OUT)],
                        yy_hbm.at[pl.ds(b * _K_OUT, _K_OUT)])


def _sc_select_sort(scores):
    B = scores.shape[0]
    mesh = plsc.VectorSubcoreMesh(core_axis_name="c", subcore_axis_name="s")
    f = pl.kernel(
        _sc_body,
        out_type=[jax.ShapeDtypeStruct((B * _K_OUT,), jnp.float32),
                  jax.ShapeDtypeStruct((B * _K_OUT,), jnp.int32),
                  jax.ShapeDtypeStruct((B * _K_OUT,), jnp.int32),
                  jax.ShapeDtypeStruct((B * _K_OUT,), jnp.int32)],
        mesh=mesh,
        compiler_params=pltpu.CompilerParams(needs_layout_passes=False),
        scratch_types=[pltpu.VMEM((_N,), jnp.float32),
                       pltpu.VMEM((_N,), jnp.int32),
                       pltpu.VMEM((_N,), jnp.float32),
                       pltpu.VMEM((_N,), jnp.int32),
                       pltpu.VMEM((16 * _NBINS,), jnp.int32)],
    )
    ss, ordr, xx, yy = f(scores.reshape(B * _N))
    xx = lax.bitcast_convert_type(xx, jnp.float32)
    yy = lax.bitcast_convert_type(yy, jnp.float32)
    return (ss.reshape(B, _K_OUT), ordr.reshape(B, _K_OUT),
            xx.reshape(B, _K_OUT), yy.reshape(B, _K_OUT))


def kernel(features, W_cls, b_cls, alpha):
    B, N, D = features.shape
    starts = [0] + [int(s) for s in np.cumsum([h * w for h, w in _LEVEL_HW])[:-1]]

    blks = [2048, 2048, 2048, 4352]
    prev_score = None
    level_scores = []
    for li, (h, w) in enumerate(_LEVEL_HW):
        n = h * w
        start = starts[li]
        blk = blks[li]
        sblk = start // blk
        nblk = -(-(start + n - sblk * blk) // blk)
        cover = nblk * blk
        off = start - sblk * blk
        upa = jnp.zeros((B, cover), dtype=jnp.float32)
        if li > 0:
            ph, pw = _LEVEL_HW[li - 1]
            up = prev_score.reshape(B, ph, pw)
            up = jnp.repeat(jnp.repeat(up, 2, axis=1), 2, axis=2) * alpha[li - 1]
            upa = lax.dynamic_update_slice(upa, up.reshape(B, n), (0, off))
        score = _level_score(features, start, n, upa, W_cls, blk) + b_cls[0]
        prev_score = score
        level_scores.append(score)

    scores = jnp.concatenate(level_scores, axis=1)
    sorted_scores, order, xx, yy = _sc_select_sort(scores)
    all_xy = jnp.stack([xx, yy], axis=-1)

    ks = [int(h * w * r) for (h, w), r in zip(_LEVEL_HW, _LEVEL_FILTER)]
    all_lvl = jnp.concatenate(
        [jnp.full((B, k), li, dtype=jnp.int32) for li, k in enumerate(ks)],
        axis=1)
    K = sum(ks)
    per_layer_idx = tuple(order[:, : int(K * r)] for r in _LAYER_FILTER)
    return (sorted_scores, all_xy, all_lvl) + per_layer_idx


# speedup vs baseline: 1.3309x; 1.3309x over previous
import jax
import jax.numpy as jnp
import numpy as np
from jax import lax
from jax.experimental import pallas as pl
from jax.experimental.pallas import tpu as pltpu, tpu_sc as plsc

_LEVEL_HW = [(16, 16), (32, 32), (64, 64), (128, 128)]
_LEVEL_FILTER = [0.25, 0.5, 1.0, 1.0]
_LAYER_FILTER = [1.0, 0.8, 0.6, 0.6, 0.4, 0.2]

_N = 21760
_CH = _N // 16
_K_OUT = 21056
_KCH = _K_OUT // 16
_I32MIN = jnp.int32(-2147483648)
_NBINS = 2048
_U = 4


def _score_body(f_ref, u_ref, w_ref, o_ref):
    fm = f_ref[0]
    mod = fm + fm * u_ref[0].reshape(-1, 1)
    o_ref[...] = jnp.dot(mod, w_ref[...],
                         preferred_element_type=jnp.float32)[:, 0][None, None]


def _level_score(features, start, n, upa, W_cls, blk=2048):
    B, N, D = features.shape
    sblk = start // blk
    nblk = -(-(start + n - sblk * blk) // blk)
    cover = nblk * blk
    up3 = upa.reshape(B * nblk, 1, blk)
    out = pl.pallas_call(
        _score_body,
        grid=(B, nblk),
        in_specs=[pl.BlockSpec((1, blk, D), lambda b, i: (b, sblk + i, 0)),
                  pl.BlockSpec((1, 1, blk), lambda b, i: (b * nblk + i, 0, 0)),
                  pl.BlockSpec((D, 1), lambda b, i: (0, 0))],
        out_specs=pl.BlockSpec((1, 1, blk), lambda b, i: (b * nblk + i, 0, 0)),
        out_shape=jax.ShapeDtypeStruct((B * nblk, 1, blk), jnp.float32),
    )(features, up3, W_cls)
    off = start - sblk * blk
    return out.reshape(B, cover)[:, off:off + n]


def _digit(vals_f32, shift):
    u = plsc.bitcast(vals_f32, jnp.int32)
    m = lax.shift_right_arithmetic(u, 31)
    key = ~(u ^ (m | _I32MIN))
    return lax.shift_right_logical(key, shift) & (_NBINS - 1)


def _sc_body(sc_hbm, ss_hbm, ord_hbm, xx_hbm, yy_hbm, Ak, Ap, Bk, Bp, hist):
    wid = lax.axis_index("s") * 2 + lax.axis_index("c")

    @pl.when(wid < 4)
    def _():
        b = wid
        lane = lax.iota(jnp.int32, 16)
        lane_str = lane * _CH
        lane_h = lane * _NBINS

        pltpu.sync_copy(sc_hbm.at[pl.ds(b * _N, _N)], Ak)

        def radix_pass(shift, Ki, Pi, Ko, Po):
            zeros16 = jnp.zeros((16,), jnp.int32)

            @plsc.parallel_loop(0, 16 * _NBINS, step=16, unroll=8)
            def _z(o):
                hist[pl.ds(o, 16)] = zeros16

            ones = jnp.ones((16,), jnp.int32)

            def pa(c, _):
                ds = [_digit(plsc.load_gather(Ki, [lane_str + (c * 8 + u)]),
                             shift)
                      for u in range(8)]
                for d in ds:
                    plsc.addupdate_scatter(hist, [lane_h + d], ones)
                return 0
            lax.fori_loop(0, _CH // 8, pa, 0)

            def sc16(dc, carry):
                vs = [hist[pl.ds(l * _NBINS + dc * 16, 16)] for l in range(16)]
                a = jnp.zeros((16,), jnp.int32)
                accs = []
                for l in range(16):
                    accs.append(a)
                    a = a + vs[l]
                total = a
                g = carry + plsc.cumsum(total) - total
                for l in range(16):
                    hist[pl.ds(l * _NBINS + dc * 16, 16)] = accs[l] + g
                return carry + jnp.sum(total, axis=0)
            plsc.parallel_loop(0, _NBINS // 16, unroll=2,
                               carry=jnp.int32(0))(sc16)

            def pb(c, _):
                idxs = [lane_str + (c * _U + u) for u in range(_U)]
                ks = [plsc.load_gather(Ki, [i]) for i in idxs]
                ps = (idxs if Pi is None
                      else [plsc.load_gather(Pi, [i]) for i in idxs])
                hs = [lane_h + _digit(k, shift) for k in ks]
                for k, p, h in zip(ks, ps, hs):
                    off = plsc.load_gather(hist, [h])
                    plsc.store_scatter(Ko, [off], k)
                    plsc.store_scatter(Po, [off], p)
                    plsc.store_scatter(hist, [h], off + 1)
                return 0
            lax.fori_loop(0, _CH // _U, pb, 0)

        radix_pass(0, Ak, None, Bk, Bp)
        radix_pass(11, Bk, Bp, Ak, Ap)
        radix_pass(22, Ak, Ap, Bk, Bp)

        zero = jnp.int32(0)

        def post(c, carry):
            for u in range(2):
                gsel, l0, l1, l2, l3 = carry
                cc = c * 2 + u
                s = Bk[pl.ds(cc * 16, 16)]
                t = Bp[pl.ds(cc * 16, 16)]
                ge1 = t >= 256
                ge2 = t >= 1280
                ge3 = t >= 5376
                sh = (ge1.astype(jnp.int32) + ge2.astype(jnp.int32)
                      + ge3.astype(jnp.int32)) << 3
                enc = lax.shift_left(jnp.ones((16,), jnp.int32), sh)
                scs = plsc.cumsum(enc)
                cnt = lax.shift_right_logical(scs - enc, sh) & 255
                lb = jnp.where(ge2, jnp.where(ge3, l3, l2),
                               jnp.where(ge1, l1, l0))
                rank = lb + cnt
                kv = jnp.where(ge2, jnp.where(ge3, 16384, 4096),
                               jnp.where(ge1, 512, 64))
                offv = jnp.where(ge2, jnp.where(ge3, 4672, 576),
                                 jnp.where(ge1, 64, 0))
                sel = rank < kv
                seli = sel.astype(jnp.int32)
                sx = plsc.cumsum(seli)
                gr = gsel + sx - seli
                gidx = jnp.where(sel, gr, zero)
                concat = offv + rank
                cidx = jnp.where(sel, concat, zero)
                plsc.store_scatter(Bk, [gidx], s, mask=sel)
                plsc.store_scatter(Ap, [gidx], concat, mask=sel)
                plsc.store_scatter(Ak, [cidx], plsc.bitcast(t, jnp.float32),
                                   mask=sel)
                tot = jnp.sum(enc, axis=0)
                nsel = jnp.sum(seli, axis=0)
                carry = (gsel + nsel,
                         l0 + (tot & 255),
                         l1 + (lax.shift_right_logical(tot, 8) & 255),
                         l2 + (lax.shift_right_logical(tot, 16) & 255),
                         l3 + (lax.shift_right_logical(tot, 24) & 255))
            return carry
        lax.fori_loop(0, _CH // 2, post, (zero, zero, zero, zero, zero))

        pltpu.sync_copy(Bk.at[pl.ds(0, _K_OUT)],
                        ss_hbm.at[pl.ds(b * _K_OUT, _K_OUT)])
        pltpu.sync_copy(Ap.at[pl.ds(0, _K_OUT)],
                        ord_hbm.at[pl.ds(b * _K_OUT, _K_OUT)])

        half = jnp.float32(0.5)
        onesv = jnp.ones((16,), jnp.int32)

        @plsc.parallel_loop(0, _KCH, unroll=4)
        def _xy(cc):
                tb = plsc.bitcast(Ak[pl.ds(cc * 16, 16)], jnp.int32)
                q = cc * 16 + lane
                qge1 = q >= 64
                qge2 = q >= 576
                qge3 = q >= 4672
                lvlq = (qge1.astype(jnp.int32) + qge2.astype(jnp.int32)
                        + qge3.astype(jnp.int32))
                startv = jnp.where(qge2, jnp.where(qge3, 5376, 1280),
                                   jnp.where(qge1, 256, 0))
                logw = 4 + lvlq
                uu = tb - startv
                jx = uu & (lax.shift_left(onesv, logw) - 1)
                iy = lax.shift_right_logical(uu, logw)
                invw = plsc.bitcast(lax.shift_left(127 - logw, 23),
                                    jnp.float32)
                xv = (jx.astype(jnp.float32) + half) * invw
                yv = (iy.astype(jnp.float32) + half) * invw
                Ap[pl.ds(cc * 16, 16)] = plsc.bitcast(xv, jnp.int32)
                Bp[pl.ds(cc * 16, 16)] = plsc.bitcast(yv, jnp.int32)

        pltpu.sync_copy(Ap.at[pl.ds(0, _K_OUT)],
                        xx_hbm.at[pl.ds(b * _K_OUT, _K_OUT)])
        pltpu.sync_copy(Bp.at[pl.ds(0, _K_OUT)],
                        yy_hbm.at[pl.ds(b * _K_OUT, _K_OUT)])


def _sc_select_sort(scores):
    B = scores.shape[0]
    mesh = plsc.VectorSubcoreMesh(core_axis_name="c", subcore_axis_name="s")
    f = pl.kernel(
        _sc_body,
        out_type=[jax.ShapeDtypeStruct((B * _K_OUT,), jnp.float32),
                  jax.ShapeDtypeStruct((B * _K_OUT,), jnp.int32),
                  jax.ShapeDtypeStruct((B * _K_OUT,), jnp.int32),
                  jax.ShapeDtypeStruct((B * _K_OUT,), jnp.int32)],
        mesh=mesh,
        compiler_params=pltpu.CompilerParams(needs_layout_passes=False),
        scratch_types=[pltpu.VMEM((_N,), jnp.float32),
                       pltpu.VMEM((_N,), jnp.int32),
                       pltpu.VMEM((_N,), jnp.float32),
                       pltpu.VMEM((_N,), jnp.int32),
                       pltpu.VMEM((16 * _NBINS,), jnp.int32)],
    )
    ss, ordr, xx, yy = f(scores.reshape(B * _N))
    xx = lax.bitcast_convert_type(xx, jnp.float32)
    yy = lax.bitcast_convert_type(yy, jnp.float32)
    return (ss.reshape(B, _K_OUT), ordr.reshape(B, _K_OUT),
            xx.reshape(B, _K_OUT), yy.reshape(B, _K_OUT))


def kernel(features, W_cls, b_cls, alpha):
    B, N, D = features.shape
    starts = [0] + [int(s) for s in np.cumsum([h * w for h, w in _LEVEL_HW])[:-1]]

    blks = [2048, 2048, 2048, 4352]
    prev_score = None
    level_scores = []
    for li, (h, w) in enumerate(_LEVEL_HW):
        n = h * w
        start = starts[li]
        blk = blks[li]
        sblk = start // blk
        nblk = -(-(start + n - sblk * blk) // blk)
        cover = nblk * blk
        off = start - sblk * blk
        upa = jnp.zeros((B, cover), dtype=jnp.float32)
        if li > 0:
            ph, pw = _LEVEL_HW[li - 1]
            up = prev_score.reshape(B, ph, pw)
            up = jnp.repeat(jnp.repeat(up, 2, axis=1), 2, axis=2) * alpha[li - 1]
            upa = lax.dynamic_update_slice(upa, up.reshape(B, n), (0, off))
        score = _level_score(features, start, n, upa, W_cls, blk) + b_cls[0]
        prev_score = score
        level_scores.append(score)

    scores = jnp.concatenate(level_scores, axis=1)
    sorted_scores, order, xx, yy = _sc_select_sort(scores)
    all_xy = jnp.stack([xx, yy], axis=-1)

    ks = [int(h * w * r) for (h, w), r in zip(_LEVEL_HW, _LEVEL_FILTER)]
    all_lvl = jnp.concatenate(
        [jnp.full((B, k), li, dtype=jnp.int32) for li, k in enumerate(ks)],
        axis=1)
    K = sum(ks)
    per_layer_idx = tuple(order[:, : int(K * r)] for r in _LAYER_FILTER)
    return (sorted_scores, all_xy, all_lvl) + per_layer_idx
